# baseline (speedup 1.0000x reference)
import jax
import jax.numpy as jnp
from jax.experimental import pallas as pl


def _sage(x, src, dst, Wl, bl, Wr, n):
    msg = x[src]
    agg = jax.ops.segment_sum(msg, dst, num_segments=n)
    cnt = jax.ops.segment_sum(jnp.ones((src.shape[0], 1), x.dtype), dst, num_segments=n)
    mean = agg / jnp.maximum(cnt, 1.0)
    return mean @ Wl.T + bl + x @ Wr.T


def kernel(x, edge_index, W1l, b1l, W1r, W2l, b2l, W2r, Wf1, bf1, Wf2, bf2):
    src = edge_index[0]
    dst = edge_index[1]
    n = x.shape[0]
    h = jax.nn.relu(_sage(x, src, dst, W1l, b1l, W1r, n))
    h = jax.nn.relu(_sage(h, src, dst, W2l, b2l, W2r, n))
    g = jnp.mean(h, axis=0, keepdims=True)
    z = jax.nn.relu(g @ Wf1.T + bf1)
    out = z @ Wf2.T + bf2
    return jax.nn.sigmoid(out).squeeze(-1)


# speedup vs baseline: 1.7305x; 1.7305x over previous
import functools

import jax
import jax.numpy as jnp
from jax import lax
from jax.experimental import pallas as pl
from jax.experimental.pallas import tpu as pltpu
from jax.experimental.pallas import tpu_sc as plsc

N = 10000
NP = 10240
E = 160000
EP = 163840
D = 256
H = 512
NSUB = 16
NCORE = 2
EC = EP // (NSUB * NCORE) // 128
ROWS_PER_TILE = NP // NSUB
W = 128
JUNK_ROW = N + 16


def _fill(buf, val):
    def frow(r, _):
        def fcol(q, _):
            buf[r, pl.ds(q * 16, 16)] = jnp.full((16,), val, jnp.float32)
            return 0
        return lax.fori_loop(0, W // 16, fcol, 0)
    lax.fori_loop(0, 128, frow, 0)


def _make_sc_agg(n_tables, count_chunk):
    n_chunks = n_tables + (1 if count_chunk else 0)
    mesh = plsc.VectorSubcoreMesh(core_axis_name="c", subcore_axis_name="s")

    def body(*refs):
        tabs = refs[:n_tables]
        src_hbm = refs[n_tables]
        dst_hbm = refs[n_tables + 1]
        out = refs[n_tables + 2]
        src_v, dst_v, gbuf, zbuf, acc, sem = refs[n_tables + 3:]

        c = lax.axis_index("c")
        s = lax.axis_index("s")
        wid = c * NSUB + s
        row0 = s * ROWS_PER_TILE

        pltpu.sync_copy(src_hbm.at[wid], src_v)
        pltpu.sync_copy(dst_hbm.at[wid], dst_v)

        _fill(zbuf, 0.0)
        if count_chunk:
            _fill(gbuf, 1.0)

        for k in range(n_chunks):
            for r in range(ROWS_PER_TILE // 128):
                pltpu.sync_copy(zbuf, acc.at[pl.ds(row0 + r * 128, 128)])
            plsc.subcore_barrier()
            if count_chunk and k == 0:
                def cbody(j, _):
                    pltpu.sync_copy(gbuf, acc.at[dst_v.at[j]], add=True)
                    return 0
                lax.fori_loop(0, EC, cbody, 0)
            else:
                tab = tabs[k - 1 if count_chunk else k]
                def ebody(j, _):
                    pltpu.async_copy(tab.at[src_v.at[j]], gbuf, sem).wait()
                    pltpu.sync_copy(gbuf, acc.at[dst_v.at[j]], add=True)
                    return 0
                lax.fori_loop(0, EC, ebody, 0)
            plsc.subcore_barrier()
            for r in range(ROWS_PER_TILE // 128):
                sl = pl.ds(row0 + r * 128, 128)
                pltpu.sync_copy(acc.at[sl], out.at[c, k, sl])

    return pl.kernel(
        body,
        out_type=jax.ShapeDtypeStruct((NCORE, n_chunks, NP, W), jnp.float32),
        mesh=mesh,
        scratch_types=[
            pltpu.VMEM((EC, 128), jnp.int32),
            pltpu.VMEM((EC, 128), jnp.int32),
            pltpu.VMEM((128, W), jnp.float32),
            pltpu.VMEM((128, W), jnp.float32),
            pltpu.VMEM_SHARED((NP, W), jnp.float32),
            pltpu.SemaphoreType.DMA,
        ],
    )


_sc_agg1 = _make_sc_agg(2, True)
_sc_agg2 = _make_sc_agg(4, False)


_DOT = functools.partial(jax.lax.dot_general,
                         dimension_numbers=(((1,), (0,)), ((), ())),
                         preferred_element_type=jnp.float32,
                         precision=jax.lax.Precision.HIGHEST)


def _mm1_body(p_ref, x_ref, w_ref, b_ref, h0, h1o, h2o, h3o):
    p = p_ref[...]
    cnt = p[0, 0] + p[1, 0]
    inv = 1.0 / jnp.maximum(cnt, 1.0)
    acc = _DOT((p[0, 1] + p[1, 1]) * inv, w_ref[0:128, :])
    acc += _DOT((p[0, 2] + p[1, 2]) * inv, w_ref[128:256, :])
    acc += _DOT(x_ref[...], w_ref[256:512, :])
    h = jnp.maximum(acc + b_ref[...], 0.0)
    h0[...] = h[:, 0:128]
    h1o[...] = h[:, 128:256]
    h2o[...] = h[:, 256:384]
    h3o[...] = h[:, 384:512]


def _mm2_body(q_ref, pc_ref, h0, h1r, h2r, h3r, w2_ref, b2_ref,
              wf1_ref, bf1_ref, wf2_ref, bf2_ref, out_ref, acc_ref):
    i = pl.program_id(0)
    q = q_ref[...]
    pc = pc_ref[...]
    inv = 1.0 / jnp.maximum(pc[0, 0] + pc[1, 0], 1.0)
    hrefs = (h0, h1r, h2r, h3r)
    acc = jnp.zeros((256, H), jnp.float32)
    for k in range(4):
        acc += _DOT((q[0, k] + q[1, k]) * inv, w2_ref[k * 128:(k + 1) * 128, :])
        acc += _DOT(hrefs[k][...], w2_ref[H + k * 128:H + (k + 1) * 128, :])
    h2 = jnp.maximum(acc + b2_ref[...], 0.0)
    row = i * 256 + lax.broadcasted_iota(jnp.int32, (256, 1), 0)
    h2 = jnp.where(row < N, h2, 0.0)
    part = jnp.sum(h2, axis=0, keepdims=True)

    @pl.when(i == 0)
    def _():
        acc_ref[...] = part

    @pl.when(i > 0)
    def _():
        acc_ref[...] = acc_ref[...] + part

    @pl.when(i == (NP // 256) - 1)
    def _():
        g = acc_ref[...] * (1.0 / N)
        z = jnp.maximum(_DOT(g, wf1_ref[...]) + bf1_ref[...], 0.0)
        o = _DOT(z, wf2_ref[...])
        logit = jnp.sum(o, axis=1, keepdims=True) + bf2_ref[...]
        out_ref[...] = 1.0 / (1.0 + jnp.exp(-logit))


def kernel(x, edge_index, W1l, b1l, W1r, W2l, b2l, W2r, Wf1, bf1, Wf2, bf2):
    f32 = jnp.float32
    src = edge_index[0].astype(jnp.int32)
    dst = edge_index[1].astype(jnp.int32)
    srcp = jnp.concatenate([src, jnp.zeros((EP - E,), jnp.int32)]).reshape(32, EC, 128)
    dstp = jnp.concatenate([dst, jnp.full((EP - E,), JUNK_ROW, jnp.int32)]).reshape(32, EC, 128)

    xp = jnp.pad(x.astype(f32), ((0, NP - N), (0, 0)))
    xtA = xp[:, :128]
    xtB = xp[:, 128:]

    p1 = _sc_agg1(xtA, xtB, srcp, dstp)

    w1cat = jnp.concatenate([W1l.T, W1r.T], axis=0)
    grid = (NP // 256,)
    h_sh = jax.ShapeDtypeStruct((NP, 128), f32)
    h0, h1c, h2c, h3c = pl.pallas_call(
        _mm1_body,
        grid=grid,
        in_specs=[
            pl.BlockSpec((2, 3, 256, 128), lambda i: (0, 0, i, 0)),
            pl.BlockSpec((256, D), lambda i: (i, 0)),
            pl.BlockSpec((2 * D, H), lambda i: (0, 0)),
            pl.BlockSpec((1, H), lambda i: (0, 0)),
        ],
        out_specs=[pl.BlockSpec((256, 128), lambda i: (i, 0))] * 4,
        out_shape=[h_sh, h_sh, h_sh, h_sh],
    )(p1, xp, w1cat, b1l.reshape(1, H))

    p2 = _sc_agg2(h0, h1c, h2c, h3c, srcp, dstp)

    w2cat = jnp.concatenate([W2l.T, W2r.T], axis=0)
    wf2p = jnp.concatenate([Wf2.T, jnp.zeros((H // 2, 127), f32)], axis=1)
    out = pl.pallas_call(
        _mm2_body,
        grid=grid,
        in_specs=[
            pl.BlockSpec((2, 4, 256, 128), lambda i: (0, 0, i, 0)),
            pl.BlockSpec((2, 1, 256, 128), lambda i: (0, 0, i, 0)),
            pl.BlockSpec((256, 128), lambda i: (i, 0)),
            pl.BlockSpec((256, 128), lambda i: (i, 0)),
            pl.BlockSpec((256, 128), lambda i: (i, 0)),
            pl.BlockSpec((256, 128), lambda i: (i, 0)),
            pl.BlockSpec((2 * H, H), lambda i: (0, 0)),
            pl.BlockSpec((1, H), lambda i: (0, 0)),
            pl.BlockSpec((H, H // 2), lambda i: (0, 0)),
            pl.BlockSpec((1, H // 2), lambda i: (0, 0)),
            pl.BlockSpec((H // 2, 128), lambda i: (0, 0)),
            pl.BlockSpec((1, 1), lambda i: (0, 0)),
        ],
        out_specs=pl.BlockSpec((1, 1), lambda i: (0, 0)),
        out_shape=jax.ShapeDtypeStruct((1, 1), f32),
        scratch_shapes=[pltpu.VMEM((1, H), f32)],
    )(p2, p1[:, 0:1], h0, h1c, h2c, h3c, w2cat, b2l.reshape(1, H),
      Wf1.T, bf1.reshape(1, H // 2), wf2p, bf2.reshape(1, 1))
    return out.reshape(1)


# speedup vs baseline: 1.9353x; 1.1183x over previous
import functools

import jax
import jax.numpy as jnp
from jax import lax
from jax.experimental import pallas as pl
from jax.experimental.pallas import tpu as pltpu
from jax.experimental.pallas import tpu_sc as plsc

N = 10000
NP = 10240
E = 160000
EP = 163840
D = 256
H = 512
NSUB = 16
NCORE = 2
EC = EP // (NSUB * NCORE) // 128
ROWS_PER_TILE = NP // NSUB
W = 128
JUNK_ROW = N + 16


def _fill(buf, val):
    def frow(r, _):
        def fcol(q, _):
            buf[r, pl.ds(q * 16, 16)] = jnp.full((16,), val, jnp.float32)
            return 0
        return lax.fori_loop(0, W // 16, fcol, 0)
    lax.fori_loop(0, 128, frow, 0)


def _make_sc_agg(n_tables, count_chunk):
    n_chunks = n_tables + (1 if count_chunk else 0)
    mesh = plsc.VectorSubcoreMesh(core_axis_name="c", subcore_axis_name="s")

    def body(*refs):
        tabs = refs[:n_tables]
        src_hbm = refs[n_tables]
        dst_hbm = refs[n_tables + 1]
        zeros_hbm = refs[n_tables + 2]
        out = refs[n_tables + 3]
        src_v, dst_v, b0, b1, acc, sem0, sem1 = refs[n_tables + 4:]

        c = lax.axis_index("c")
        s = lax.axis_index("s")
        wid = c * NSUB + s
        row0 = s * ROWS_PER_TILE
        slab = pl.ds(row0, ROWS_PER_TILE)

        pltpu.sync_copy(src_hbm.at[wid], src_v)
        pltpu.sync_copy(dst_hbm.at[wid], dst_v)

        if count_chunk:
            _fill(b0, 1.0)

        for k in range(n_chunks):
            pltpu.sync_copy(zeros_hbm.at[slab], acc.at[slab])
            plsc.subcore_barrier()
            if count_chunk and k == 0:
                def cbody(j, _):
                    pltpu.async_copy(b0, acc.at[dst_v.at[j]], sem0, add=True)

                    @pl.when(j >= 2)
                    def _():
                        pltpu.make_async_copy(b0, acc.at[dst_v.at[0]], sem0).wait()
                    return 0
                lax.fori_loop(0, EC, cbody, 0)
                pltpu.make_async_copy(b0, acc.at[dst_v.at[0]], sem0).wait()
                pltpu.make_async_copy(b0, acc.at[dst_v.at[0]], sem0).wait()
            else:
                tab = tabs[k - 1 if count_chunk else k]
                pltpu.async_copy(tab.at[src_v.at[0]], b0, sem0)

                def ebody(jj, _):
                    j0 = 2 * jj
                    pltpu.async_copy(tab.at[src_v.at[j0 + 1]], b1, sem1)
                    pltpu.make_async_copy(tab.at[src_v.at[j0]], b0, sem0).wait()
                    pltpu.sync_copy(b0, acc.at[dst_v.at[j0]], add=True)

                    @pl.when(jj < EC // 2 - 1)
                    def _():
                        pltpu.async_copy(tab.at[src_v.at[j0 + 2]], b0, sem0)
                    pltpu.make_async_copy(tab.at[src_v.at[j0 + 1]], b1, sem1).wait()
                    pltpu.sync_copy(b1, acc.at[dst_v.at[j0 + 1]], add=True)
                    return 0
                lax.fori_loop(0, EC // 2, ebody, 0)
            plsc.subcore_barrier()
            pltpu.sync_copy(acc.at[slab], out.at[c, k, slab])

    return pl.kernel(
        body,
        out_type=jax.ShapeDtypeStruct((NCORE, n_chunks, NP, W), jnp.float32),
        mesh=mesh,
        scratch_types=[
            pltpu.VMEM((EC, 128), jnp.int32),
            pltpu.VMEM((EC, 128), jnp.int32),
            pltpu.VMEM((128, W), jnp.float32),
            pltpu.VMEM((128, W), jnp.float32),
            pltpu.VMEM_SHARED((NP, W), jnp.float32),
            pltpu.SemaphoreType.DMA,
            pltpu.SemaphoreType.DMA,
        ],
    )


_sc_agg1 = _make_sc_agg(2, True)
_sc_agg2 = _make_sc_agg(4, False)


_DOT = functools.partial(jax.lax.dot_general,
                         dimension_numbers=(((1,), (0,)), ((), ())),
                         preferred_element_type=jnp.float32,
                         precision=jax.lax.Precision.HIGHEST)


def _mm1_body(p_ref, x_ref, w_ref, b_ref, h0, h1o, h2o, h3o):
    p = p_ref[...]
    cnt = p[0, 0] + p[1, 0]
    inv = 1.0 / jnp.maximum(cnt, 1.0)
    acc = _DOT((p[0, 1] + p[1, 1]) * inv, w_ref[0:128, :])
    acc += _DOT((p[0, 2] + p[1, 2]) * inv, w_ref[128:256, :])
    acc += _DOT(x_ref[...], w_ref[256:512, :])
    h = jnp.maximum(acc + b_ref[...], 0.0)
    h0[...] = h[:, 0:128]
    h1o[...] = h[:, 128:256]
    h2o[...] = h[:, 256:384]
    h3o[...] = h[:, 384:512]


def _mm2_body(q_ref, pc_ref, h0, h1r, h2r, h3r, w2_ref, b2_ref,
              wf1_ref, bf1_ref, wf2_ref, bf2_ref, out_ref, acc_ref):
    i = pl.program_id(0)
    q = q_ref[...]
    pc = pc_ref[...]
    inv = 1.0 / jnp.maximum(pc[0, 0] + pc[1, 0], 1.0)
    hrefs = (h0, h1r, h2r, h3r)
    acc = jnp.zeros((256, H), jnp.float32)
    for k in range(4):
        acc += _DOT((q[0, k] + q[1, k]) * inv, w2_ref[k * 128:(k + 1) * 128, :])
        acc += _DOT(hrefs[k][...], w2_ref[H + k * 128:H + (k + 1) * 128, :])
    h2 = jnp.maximum(acc + b2_ref[...], 0.0)
    row = i * 256 + lax.broadcasted_iota(jnp.int32, (256, 1), 0)
    h2 = jnp.where(row < N, h2, 0.0)
    part = jnp.sum(h2, axis=0, keepdims=True)

    @pl.when(i == 0)
    def _():
        acc_ref[...] = part

    @pl.when(i > 0)
    def _():
        acc_ref[...] = acc_ref[...] + part

    @pl.when(i == (NP // 256) - 1)
    def _():
        g = acc_ref[...] * (1.0 / N)
        z = jnp.maximum(_DOT(g, wf1_ref[...]) + bf1_ref[...], 0.0)
        o = _DOT(z, wf2_ref[...])
        logit = jnp.sum(o, axis=1, keepdims=True) + bf2_ref[...]
        out_ref[...] = 1.0 / (1.0 + jnp.exp(-logit))


def kernel(x, edge_index, W1l, b1l, W1r, W2l, b2l, W2r, Wf1, bf1, Wf2, bf2):
    f32 = jnp.float32
    src = edge_index[0].astype(jnp.int32)
    dst = edge_index[1].astype(jnp.int32)
    srcp = jnp.concatenate([src, jnp.zeros((EP - E,), jnp.int32)]).reshape(32, EC, 128)
    dstp = jnp.concatenate([dst, jnp.full((EP - E,), JUNK_ROW, jnp.int32)]).reshape(32, EC, 128)

    xp = jnp.pad(x.astype(f32), ((0, NP - N), (0, 0)))
    xtA = xp[:, :128]
    xtB = xp[:, 128:]

    zrow_hbm = jnp.zeros((NP, W), f32)
    p1 = _sc_agg1(xtA, xtB, srcp, dstp, zrow_hbm)

    w1cat = jnp.concatenate([W1l.T, W1r.T], axis=0)
    grid = (NP // 256,)
    h_sh = jax.ShapeDtypeStruct((NP, 128), f32)
    h0, h1c, h2c, h3c = pl.pallas_call(
        _mm1_body,
        grid=grid,
        in_specs=[
            pl.BlockSpec((2, 3, 256, 128), lambda i: (0, 0, i, 0)),
            pl.BlockSpec((256, D), lambda i: (i, 0)),
            pl.BlockSpec((2 * D, H), lambda i: (0, 0)),
            pl.BlockSpec((1, H), lambda i: (0, 0)),
        ],
        out_specs=[pl.BlockSpec((256, 128), lambda i: (i, 0))] * 4,
        out_shape=[h_sh, h_sh, h_sh, h_sh],
    )(p1, xp, w1cat, b1l.reshape(1, H))

    p2 = _sc_agg2(h0, h1c, h2c, h3c, srcp, dstp, zrow_hbm)

    w2cat = jnp.concatenate([W2l.T, W2r.T], axis=0)
    wf2p = jnp.concatenate([Wf2.T, jnp.zeros((H // 2, 127), f32)], axis=1)
    out = pl.pallas_call(
        _mm2_body,
        grid=grid,
        in_specs=[
            pl.BlockSpec((2, 4, 256, 128), lambda i: (0, 0, i, 0)),
            pl.BlockSpec((2, 1, 256, 128), lambda i: (0, 0, i, 0)),
            pl.BlockSpec((256, 128), lambda i: (i, 0)),
            pl.BlockSpec((256, 128), lambda i: (i, 0)),
            pl.BlockSpec((256, 128), lambda i: (i, 0)),
            pl.BlockSpec((256, 128), lambda i: (i, 0)),
            pl.BlockSpec((2 * H, H), lambda i: (0, 0)),
            pl.BlockSpec((1, H), lambda i: (0, 0)),
            pl.BlockSpec((H, H // 2), lambda i: (0, 0)),
            pl.BlockSpec((1, H // 2), lambda i: (0, 0)),
            pl.BlockSpec((H // 2, 128), lambda i: (0, 0)),
            pl.BlockSpec((1, 1), lambda i: (0, 0)),
        ],
        out_specs=pl.BlockSpec((1, 1), lambda i: (0, 0)),
        out_shape=jax.ShapeDtypeStruct((1, 1), f32),
        scratch_shapes=[pltpu.VMEM((1, H), f32)],
    )(p2, p1[:, 0:1], h0, h1c, h2c, h3c, w2cat, b2l.reshape(1, H),
      Wf1.T, bf1.reshape(1, H // 2), wf2p, bf2.reshape(1, 1))
    return out.reshape(1)
